# baseline (speedup 1.0000x reference)
import functools

import jax
import jax.numpy as jnp
from jax import lax
from jax.experimental import pallas as pl
from jax.experimental.pallas import tpu as pltpu
from jax.experimental.pallas import tpu_sc as plsc

_NC = 2
_NSW = 8
_NW = _NC * _NSW
_R, _D = 16384, 4096
_ROWS_W = _R // _NW
_CH = 16
_NB = 3
_NCHUNKS = _ROWS_W // _CH


def _ring_steps(n):
    prime = [("rs", 0), ("rs", 1)]
    steps = []
    waited = set()
    for i in range(n):
        ops = [("rw", i), ("ws", i)]
        if i + 2 <= n - 1:
            if i - 1 >= 0:
                ops.append(("ww", i - 1))
                waited.add(i - 1)
            ops.append(("rs", i + 2))
        steps.append(ops)
    drain = [("ww", i) for i in sorted(set(range(n)) - waited)]
    return prime, steps, drain


def _sc_copy_body(x_hbm, o_hbm, sp, *sems):
    c = lax.axis_index("c")
    s = lax.axis_index("s")
    wid = s * _NC + c
    base = wid * _ROWS_W
    srs, sws = sems[:_NB], sems[_NB:]

    def op(kind, i):
        b = i % _NB
        src = x_hbm.at[pl.ds(base + i * _CH, _CH)]
        dst = o_hbm.at[pl.ds(base + i * _CH, _CH)]
        if kind == "rs":
            pltpu.make_async_copy(src, sp.at[s, b], srs[b]).start()
        elif kind == "rw":
            pltpu.make_async_copy(src, sp.at[s, b], srs[b]).wait()
        elif kind == "ws":
            pltpu.make_async_copy(sp.at[s, b], dst, sws[b]).start()
        else:
            pltpu.make_async_copy(sp.at[s, b], dst, sws[b]).wait()

    prime, steps, drain = _ring_steps(_NCHUNKS)

    @pl.when(s < _NSW)
    def _():
        for kind, i in prime:
            op(kind, i)
        for ops in steps:
            for kind, i in ops:
                op(kind, i)
        for kind, i in drain:
            op(kind, i)


def kernel(x, score):
    del score
    B, S, D = x.shape
    x2 = x.reshape(_R, _D)
    mesh = plsc.VectorSubcoreMesh(core_axis_name="c", subcore_axis_name="s")
    f = functools.partial(
        pl.kernel,
        out_type=jax.ShapeDtypeStruct((_R, _D), x.dtype),
        mesh=mesh,
        scratch_types=(
            [pltpu.VMEM_SHARED((_NSW, _NB, _CH, _D), jnp.float32)]
            + [pltpu.SemaphoreType.DMA] * (2 * _NB)
        ),
    )(_sc_copy_body)
    out = f(x2)
    return out.reshape(B, S, D)


# speedup vs baseline: 1.0208x; 1.0208x over previous
import functools

import jax
import jax.numpy as jnp
from jax import lax
from jax.experimental import pallas as pl
from jax.experimental.pallas import tpu as pltpu
from jax.experimental.pallas import tpu_sc as plsc

_NC, _NS = 2, 16
_NW = _NC * _NS
_R, _D = 16384, 4096
_ROWS_W = _R // _NW
_CH = 8
_NCHUNKS = _ROWS_W // _CH


def _sc_copy_body(x_hbm, o_hbm, sp, sr0, sr1, sr2, sw0, sw1, sw2):
    c = lax.axis_index("c")
    s = lax.axis_index("s")
    wid = s * _NC + c
    base = wid * _ROWS_W
    srs, sws = (sr0, sr1, sr2), (sw0, sw1, sw2)
    n = _NCHUNKS

    def rd(i, b):
        return pltpu.make_async_copy(
            x_hbm.at[pl.ds(base + i * _CH, _CH)], sp.at[s, b], srs[b])

    def wr(i, b):
        return pltpu.make_async_copy(
            sp.at[s, b], o_hbm.at[pl.ds(base + i * _CH, _CH)], sws[b])

    rd(0, 0).start()
    rd(1, 1).start()
    rd(0, 0).wait()
    wr(0, 0).start()
    rd(2, 2).start()

    def body(j, carry):
        for b3 in range(3):
            i = 3 * j + 1 + b3
            b = (1 + b3) % 3
            pb = (b + 2) % 3
            rd(i, b).wait()
            wr(i, b).start()
            wr(i - 1, pb).wait()
            rd(i + 2, pb).start()
        return carry

    lax.fori_loop(0, (n - 4) // 3, body, 0)

    rd(n - 3, 1).wait()
    wr(n - 3, 1).start()
    wr(n - 4, 0).wait()
    rd(n - 1, 0).start()
    rd(n - 2, 2).wait()
    wr(n - 2, 2).start()
    rd(n - 1, 0).wait()
    wr(n - 1, 0).start()
    wr(n - 3, 1).wait()
    wr(n - 2, 2).wait()
    wr(n - 1, 0).wait()


def kernel(x, score):
    del score
    B, S, D = x.shape
    x2 = x.reshape(_R, _D)
    mesh = plsc.VectorSubcoreMesh(core_axis_name="c", subcore_axis_name="s")
    f = functools.partial(
        pl.kernel,
        out_type=jax.ShapeDtypeStruct((_R, _D), x.dtype),
        mesh=mesh,
        scratch_types=[
            pltpu.VMEM_SHARED((_NS, 3, _CH, _D), jnp.float32),
            pltpu.SemaphoreType.DMA,
            pltpu.SemaphoreType.DMA,
            pltpu.SemaphoreType.DMA,
            pltpu.SemaphoreType.DMA,
            pltpu.SemaphoreType.DMA,
            pltpu.SemaphoreType.DMA,
        ],
    )(_sc_copy_body)
    out = f(x2)
    return out.reshape(B, S, D)
